# Initial kernel scaffold; baseline (speedup 1.0000x reference)
import jax
import jax.numpy as jnp
from jax.experimental import pallas as pl


def kernel(reward, probs):
    raise NotImplementedError("write your pallas kernel here")


# speedup vs baseline: 79.2234x; 79.2234x over previous
import functools

import jax
import jax.numpy as jnp
import numpy as np
from jax import lax
from jax.experimental import pallas as pl
from jax.experimental.pallas import tpu as pltpu
from jax.experimental.pallas import tpu_sc as plsc

V_MIN = -10.0
V_MAX = 10.0
NUM_ATOMS = 51
ATOM_DELTA = (V_MAX - V_MIN) / (NUM_ATOMS - 1)
BS = 16384

NUM_CORES = 2
NUM_SUBCORES = 16
NUM_WORKERS = NUM_CORES * NUM_SUBCORES
ROWS_PER_W = BS // NUM_WORKERS
PAD_COLS = 64

_ATOMS_PADDED = np.asarray(
    [V_MIN + ATOM_DELTA * i for i in range(NUM_ATOMS)] + [0.0] * (PAD_COLS - NUM_ATOMS),
    dtype=np.float32,
)

_CHUNKS = ((0, False), (16, False), (32, False), (40, True))


def _sc_project(reward_hbm, probs_hbm, atoms_hbm, out_hbm,
                reward_v, probs_v, out_v, atoms_v):
    wid = lax.axis_index("s") * NUM_CORES + lax.axis_index("c")
    base = wid * ROWS_PER_W
    pltpu.sync_copy(reward_hbm.at[pl.ds(base, ROWS_PER_W)], reward_v)
    pltpu.sync_copy(probs_hbm.at[pl.ds(base, ROWS_PER_W), :], probs_v)
    pltpu.sync_copy(atoms_hbm, atoms_v)

    zeros16 = jnp.zeros((16,), jnp.float32)
    lane = lax.iota(jnp.int32, 16)
    tail_mask = (lane >= 8) & (lane <= 10)

    def row_body(r, carry):
        rvec = jnp.full((16,), r, jnp.int32)
        rwv = plsc.load_gather(reward_v, [rvec])
        for off in (0, 16, 32, 48):
            out_v[r, pl.ds(off, 16)] = zeros16
        for off, is_tail in _CHUNKS:
            av = atoms_v[pl.ds(off, 16)]
            p = probs_v[r, pl.ds(off, 16)]
            nav = jnp.clip(av + rwv, V_MIN, V_MAX)
            idxf = (nav - V_MIN) / ATOM_DELTA
            li = idxf.astype(jnp.int32)
            lif = li.astype(jnp.float32)
            eq = lif == idxf
            ui = li + jnp.where(eq, 0, 1)
            lc = ui.astype(jnp.float32) - idxf
            wu = jnp.where(eq, jnp.float32(1.0), idxf - lif)
            mask = tail_mask if is_tail else None
            plsc.addupdate_scatter(out_v, [rvec, li], p * lc, mask=mask)
            plsc.addupdate_scatter(out_v, [rvec, ui], p * wu, mask=mask)
        return carry

    lax.fori_loop(0, ROWS_PER_W, row_body, 0)
    pltpu.sync_copy(out_v, out_hbm.at[pl.ds(base, ROWS_PER_W), :])


@functools.partial(
    pl.kernel,
    out_type=jax.ShapeDtypeStruct((BS, PAD_COLS), jnp.float32),
    mesh=plsc.VectorSubcoreMesh(core_axis_name="c", subcore_axis_name="s"),
    compiler_params=pltpu.CompilerParams(
        needs_layout_passes=False, use_tc_tiling_on_sc=False),
    scratch_types=[
        pltpu.VMEM((ROWS_PER_W,), jnp.float32),
        pltpu.VMEM((ROWS_PER_W, PAD_COLS), jnp.float32),
        pltpu.VMEM((ROWS_PER_W, PAD_COLS), jnp.float32),
        pltpu.VMEM((PAD_COLS,), jnp.float32),
    ],
)
def _projection_kernel(reward_hbm, probs_hbm, atoms_hbm, out_hbm,
                       reward_v, probs_v, out_v, atoms_v):
    _sc_project(reward_hbm, probs_hbm, atoms_hbm, out_hbm,
                reward_v, probs_v, out_v, atoms_v)


def kernel(reward, probs):
    probs_p = jnp.pad(probs, ((0, 0), (0, PAD_COLS - NUM_ATOMS)))
    out_p = _projection_kernel(reward, probs_p, jnp.asarray(_ATOMS_PADDED))
    return out_p[:, :NUM_ATOMS]


# speedup vs baseline: 98.8927x; 1.2483x over previous
import functools

import jax
import jax.numpy as jnp
import numpy as np
from jax import lax
from jax.experimental import pallas as pl
from jax.experimental.pallas import tpu as pltpu
from jax.experimental.pallas import tpu_sc as plsc

V_MIN = -10.0
V_MAX = 10.0
NUM_ATOMS = 51
ATOM_DELTA = (V_MAX - V_MIN) / (NUM_ATOMS - 1)
INV_DELTA = 1.0 / ATOM_DELTA
BS = 16384

NUM_CORES = 2
NUM_SUBCORES = 16
NUM_WORKERS = NUM_CORES * NUM_SUBCORES
ROWS_PER_W = BS // NUM_WORKERS
_ATOMS_SCALED = np.asarray(
    [np.float32(np.float32(V_MIN + ATOM_DELTA * i) * INV_DELTA + 50.0 / 2.0)
     for i in range(NUM_ATOMS)],
    dtype=np.float32,
)

_CHUNKS = ((0, False), (16, False), (32, False), (35, True))
_TOP_BIN = NUM_ATOMS - 1


def _sc_project(rws_hbm, probs_hbm, atoms_hbm, out_hbm,
                rws_v, probs_v, out_v, atoms_v):
    wid = lax.axis_index("s") * NUM_CORES + lax.axis_index("c")
    base = wid * ROWS_PER_W
    pltpu.sync_copy(rws_hbm.at[pl.ds(base, ROWS_PER_W)], rws_v)
    pltpu.sync_copy(probs_hbm.at[pl.ds(base, ROWS_PER_W), :], probs_v)
    pltpu.sync_copy(atoms_hbm, atoms_v)

    zeros16 = jnp.zeros((16,), jnp.float32)
    lane = lax.iota(jnp.int32, 16)
    tail_mask = lane >= 13
    avs = [atoms_v[pl.ds(off, 16)] for off, _ in _CHUNKS]

    def row_body(r, carry):
        rvec = jnp.full((16,), r, jnp.int32)
        rwv = plsc.load_gather(rws_v, [rvec])
        for off in (0, 16, 32, 35):
            out_v[r, pl.ds(off, 16)] = zeros16
        for (off, is_tail), av in zip(_CHUNKS, avs):
            p = probs_v[r, pl.ds(off, 16)]
            idxf = jnp.clip(av + rwv, 0.0, float(_TOP_BIN))
            li = idxf.astype(jnp.int32)
            f = idxf - li.astype(jnp.float32)
            ui = jnp.minimum(li + 1, _TOP_BIN)
            mask = tail_mask if is_tail else None
            plsc.addupdate_scatter(out_v, [rvec, li], p * (1.0 - f), mask=mask)
            plsc.addupdate_scatter(out_v, [rvec, ui], p * f, mask=mask)
        return carry

    lax.fori_loop(0, ROWS_PER_W, row_body, 0, unroll=4)
    pltpu.sync_copy(out_v, out_hbm.at[pl.ds(base, ROWS_PER_W), :])


@functools.partial(
    pl.kernel,
    out_type=jax.ShapeDtypeStruct((BS, NUM_ATOMS), jnp.float32),
    mesh=plsc.VectorSubcoreMesh(core_axis_name="c", subcore_axis_name="s"),
    compiler_params=pltpu.CompilerParams(
        needs_layout_passes=False, use_tc_tiling_on_sc=False),
    scratch_types=[
        pltpu.VMEM((ROWS_PER_W,), jnp.float32),
        pltpu.VMEM((ROWS_PER_W, NUM_ATOMS), jnp.float32),
        pltpu.VMEM((ROWS_PER_W, NUM_ATOMS), jnp.float32),
        pltpu.VMEM((NUM_ATOMS,), jnp.float32),
    ],
)
def _projection_kernel(rws_hbm, probs_hbm, atoms_hbm, out_hbm,
                       rws_v, probs_v, out_v, atoms_v):
    _sc_project(rws_hbm, probs_hbm, atoms_hbm, out_hbm,
                rws_v, probs_v, out_v, atoms_v)


def kernel(reward, probs):
    rws = reward * jnp.float32(INV_DELTA)
    return _projection_kernel(rws, probs, jnp.asarray(_ATOMS_SCALED))


# speedup vs baseline: 144.2970x; 1.4591x over previous
import functools

import jax
import jax.numpy as jnp
import numpy as np
from jax import lax
from jax.experimental import pallas as pl
from jax.experimental.pallas import tpu as pltpu
from jax.experimental.pallas import tpu_sc as plsc

V_MIN = -10.0
V_MAX = 10.0
NUM_ATOMS = 51
ATOM_DELTA = (V_MAX - V_MIN) / (NUM_ATOMS - 1)
INV_DELTA = 1.0 / ATOM_DELTA
BS = 16384

NUM_CORES = 2
NUM_SUBCORES = 16
NUM_WORKERS = NUM_CORES * NUM_SUBCORES
ROWS_PER_W = BS // NUM_WORKERS
_ATOMS_SCALED = np.asarray(
    [np.float32(np.float32(V_MIN + ATOM_DELTA * i) * INV_DELTA + 50.0 / 2.0)
     for i in range(NUM_ATOMS)],
    dtype=np.float32,
)

_CHUNKS = ((0, False), (16, False), (32, False), (35, True))
_TOP_BIN = NUM_ATOMS - 1


def _sc_project(rws_hbm, probs_hbm, atoms_hbm, out_hbm,
                rws_v, work_v, atoms_v):
    wid = lax.axis_index("s") * NUM_CORES + lax.axis_index("c")
    base = wid * ROWS_PER_W
    pltpu.sync_copy(rws_hbm.at[pl.ds(base, ROWS_PER_W)], rws_v)
    pltpu.sync_copy(probs_hbm.at[pl.ds(base, ROWS_PER_W), :], work_v)
    pltpu.sync_copy(atoms_hbm, atoms_v)

    zeros16 = jnp.zeros((16,), jnp.float32)
    lane = lax.iota(jnp.int32, 16)
    tail_mask = lane >= 13
    avs = [atoms_v[pl.ds(off, 16)] for off, _ in _CHUNKS]

    def row_body(r, carry):
        rvec = jnp.full((16,), r, jnp.int32)
        rwv = plsc.load_gather(rws_v, [rvec])
        ps = [work_v[r, pl.ds(off, 16)] for off, _ in _CHUNKS]
        for off in (0, 16, 32, 35):
            work_v[r, pl.ds(off, 16)] = zeros16
        for (off, is_tail), av, p in zip(_CHUNKS, avs, ps):
            idxf = jnp.clip(av + rwv, 0.0, float(_TOP_BIN))
            li = idxf.astype(jnp.int32)
            f = idxf - li.astype(jnp.float32)
            ui = jnp.minimum(li + 1, _TOP_BIN)
            mask = tail_mask if is_tail else None
            plsc.addupdate_scatter(work_v, [rvec, li], p * (1.0 - f), mask=mask)
            plsc.addupdate_scatter(work_v, [rvec, ui], p * f, mask=mask)
        return carry

    lax.fori_loop(0, ROWS_PER_W, row_body, 0, unroll=4)
    pltpu.sync_copy(work_v, out_hbm.at[pl.ds(base, ROWS_PER_W), :])


@functools.partial(
    pl.kernel,
    out_type=jax.ShapeDtypeStruct((BS, NUM_ATOMS), jnp.float32),
    mesh=plsc.VectorSubcoreMesh(core_axis_name="c", subcore_axis_name="s"),
    compiler_params=pltpu.CompilerParams(
        needs_layout_passes=False, use_tc_tiling_on_sc=True),
    scratch_types=[
        pltpu.VMEM((ROWS_PER_W,), jnp.float32),
        pltpu.VMEM((ROWS_PER_W, NUM_ATOMS), jnp.float32),
        pltpu.VMEM((NUM_ATOMS,), jnp.float32),
    ],
)
def _projection_kernel(rws_hbm, probs_hbm, atoms_hbm, out_hbm,
                       rws_v, work_v, atoms_v):
    _sc_project(rws_hbm, probs_hbm, atoms_hbm, out_hbm,
                rws_v, work_v, atoms_v)


def kernel(reward, probs):
    rws = reward * jnp.float32(INV_DELTA)
    return _projection_kernel(rws, probs, jnp.asarray(_ATOMS_SCALED))


# speedup vs baseline: 225.0655x; 1.5597x over previous
import functools

import jax
import jax.numpy as jnp
from jax import lax
from jax.experimental import pallas as pl
from jax.experimental.pallas import tpu as pltpu
from jax.experimental.pallas import tpu_sc as plsc

V_MIN = -10.0
V_MAX = 10.0
NUM_ATOMS = 51
ATOM_DELTA = (V_MAX - V_MIN) / (NUM_ATOMS - 1)
INV_DELTA = 1.0 / ATOM_DELTA
BS = 16384
TOP_BIN = NUM_ATOMS - 1

NUM_CORES = 2
NUM_SUBCORES = 16
NUM_WORKERS = NUM_CORES * NUM_SUBCORES
COLS_PER_W = BS // NUM_WORKERS
GROUPS = COLS_PER_W // 16


def _sc_project(rw_hbm, probs_t_hbm, out_t_hbm, rws_v, probs_v, out_v):
    wid = lax.axis_index("s") * NUM_CORES + lax.axis_index("c")
    base = wid * COLS_PER_W
    pltpu.sync_copy(rw_hbm.at[pl.ds(base, COLS_PER_W)], rws_v)
    pltpu.sync_copy(probs_t_hbm.at[:, pl.ds(base, COLS_PER_W)], probs_v)

    zeros16 = jnp.zeros((16,), jnp.float32)
    lane = lax.iota(jnp.int32, 16)

    def zero_body(j, carry):
        for g in range(GROUPS):
            out_v[j, pl.ds(g * 16, 16)] = zeros16
        return carry

    lax.fori_loop(0, NUM_ATOMS, zero_body, 0, unroll=2)

    def group_body(g, carry):
        col0 = g * 16
        cvec = lane + col0
        rws16 = rws_v[pl.ds(col0, 16)] * jnp.float32(INV_DELTA)
        for j in range(NUM_ATOMS):
            idxf = jnp.clip(jnp.float32(j) + rws16, 0.0, float(TOP_BIN))
            li = idxf.astype(jnp.int32)
            f = idxf - li.astype(jnp.float32)
            ui = jnp.minimum(li + 1, TOP_BIN)
            p = probs_v[j, pl.ds(col0, 16)]
            plsc.addupdate_scatter(out_v, [li, cvec], p * (1.0 - f))
            plsc.addupdate_scatter(out_v, [ui, cvec], p * f)
        return carry

    lax.fori_loop(0, GROUPS, group_body, 0)
    pltpu.sync_copy(out_v, out_t_hbm.at[:, pl.ds(base, COLS_PER_W)])


@functools.partial(
    pl.kernel,
    out_type=jax.ShapeDtypeStruct((NUM_ATOMS, BS), jnp.float32),
    mesh=plsc.VectorSubcoreMesh(core_axis_name="c", subcore_axis_name="s"),
    compiler_params=pltpu.CompilerParams(
        needs_layout_passes=False, use_tc_tiling_on_sc=True),
    scratch_types=[
        pltpu.VMEM((COLS_PER_W,), jnp.float32),
        pltpu.VMEM((NUM_ATOMS, COLS_PER_W), jnp.float32),
        pltpu.VMEM((NUM_ATOMS, COLS_PER_W), jnp.float32),
    ],
)
def _projection_kernel(rw_hbm, probs_t_hbm, out_t_hbm, rws_v, probs_v, out_v):
    _sc_project(rw_hbm, probs_t_hbm, out_t_hbm, rws_v, probs_v, out_v)


def kernel(reward, probs):
    return _projection_kernel(reward, probs.T).T


# speedup vs baseline: 227.4622x; 1.0106x over previous
import functools

import jax
import jax.numpy as jnp
from jax import lax
from jax.experimental import pallas as pl
from jax.experimental.pallas import tpu as pltpu
from jax.experimental.pallas import tpu_sc as plsc

V_MIN = -10.0
V_MAX = 10.0
NUM_ATOMS = 51
ATOM_DELTA = (V_MAX - V_MIN) / (NUM_ATOMS - 1)
INV_DELTA = 1.0 / ATOM_DELTA
BS = 16384
TOP_BIN = NUM_ATOMS - 1

NUM_CORES = 2
NUM_SUBCORES = 16
NUM_WORKERS = NUM_CORES * NUM_SUBCORES
COLS_PER_W = BS // NUM_WORKERS
GROUPS = COLS_PER_W // 16


def _sc_project(rw_hbm, probs_t_hbm, out_t_hbm, rws_v, probs_v, out_v, sem):
    wid = lax.axis_index("s") * NUM_CORES + lax.axis_index("c")
    base = wid * COLS_PER_W
    rw_cp = pltpu.async_copy(rw_hbm.at[pl.ds(base, COLS_PER_W)], rws_v, sem)
    p_cp = pltpu.async_copy(
        probs_t_hbm.at[:, pl.ds(base, COLS_PER_W)], probs_v, sem)

    zeros16 = jnp.zeros((16,), jnp.float32)
    lane = lax.iota(jnp.int32, 16)

    def zero_body(j, carry):
        for g in range(GROUPS):
            out_v[j, pl.ds(g * 16, 16)] = zeros16
        return carry

    lax.fori_loop(0, NUM_ATOMS, zero_body, 0, unroll=2)
    rw_cp.wait()
    p_cp.wait()

    def group_body(g, carry):
        col0 = g * 16
        cvec = lane + col0
        rws16 = rws_v[pl.ds(col0, 16)] * jnp.float32(INV_DELTA)
        for j in range(NUM_ATOMS):
            idxf = jnp.clip(jnp.float32(j) + rws16, 0.0, float(TOP_BIN))
            li = idxf.astype(jnp.int32)
            f = idxf - li.astype(jnp.float32)
            ui = jnp.minimum(li + 1, TOP_BIN)
            p = probs_v[j, pl.ds(col0, 16)]
            plsc.addupdate_scatter(out_v, [li, cvec], p * (1.0 - f))
            plsc.addupdate_scatter(out_v, [ui, cvec], p * f)
        return carry

    lax.fori_loop(0, GROUPS, group_body, 0, unroll=2)
    pltpu.sync_copy(out_v, out_t_hbm.at[:, pl.ds(base, COLS_PER_W)])


@functools.partial(
    pl.kernel,
    out_type=jax.ShapeDtypeStruct((NUM_ATOMS, BS), jnp.float32),
    mesh=plsc.VectorSubcoreMesh(core_axis_name="c", subcore_axis_name="s"),
    compiler_params=pltpu.CompilerParams(
        needs_layout_passes=False, use_tc_tiling_on_sc=True),
    scratch_types=[
        pltpu.VMEM((COLS_PER_W,), jnp.float32),
        pltpu.VMEM((NUM_ATOMS, COLS_PER_W), jnp.float32),
        pltpu.VMEM((NUM_ATOMS, COLS_PER_W), jnp.float32),
        pltpu.SemaphoreType.DMA,
    ],
)
def _projection_kernel(rw_hbm, probs_t_hbm, out_t_hbm,
                       rws_v, probs_v, out_v, sem):
    _sc_project(rw_hbm, probs_t_hbm, out_t_hbm, rws_v, probs_v, out_v, sem)


def kernel(reward, probs):
    return _projection_kernel(reward, probs.T).T


# speedup vs baseline: 229.1847x; 1.0076x over previous
import functools

import jax
import jax.numpy as jnp
from jax import lax
from jax.experimental import pallas as pl
from jax.experimental.pallas import tpu as pltpu
from jax.experimental.pallas import tpu_sc as plsc

V_MIN = -10.0
V_MAX = 10.0
NUM_ATOMS = 51
ATOM_DELTA = (V_MAX - V_MIN) / (NUM_ATOMS - 1)
INV_DELTA = 1.0 / ATOM_DELTA
BS = 16384
TOP_BIN = NUM_ATOMS - 1

NUM_CORES = 2
NUM_SUBCORES = 16
NUM_WORKERS = NUM_CORES * NUM_SUBCORES
COLS_PER_W = BS // NUM_WORKERS
GROUPS = COLS_PER_W // 16


def _sc_project(rw_hbm, probs_t_hbm, out_t_hbm, rws_v, probs_v, out_v, sem):
    wid = lax.axis_index("s") * NUM_CORES + lax.axis_index("c")
    base = wid * COLS_PER_W
    rw_cp = pltpu.async_copy(rw_hbm.at[pl.ds(base, COLS_PER_W)], rws_v, sem)
    p_cp = pltpu.async_copy(
        probs_t_hbm.at[:, pl.ds(base, COLS_PER_W)], probs_v, sem)

    zeros16 = jnp.zeros((16,), jnp.float32)
    lane = lax.iota(jnp.int32, 16)

    @plsc.parallel_loop(0, NUM_ATOMS, step=1, unroll=2)
    def zero_body(j):
        for g in range(GROUPS):
            out_v[j, pl.ds(g * 16, 16)] = zeros16

    rw_cp.wait()
    p_cp.wait()

    @plsc.parallel_loop(0, GROUPS, step=1, unroll=2)
    def group_body(g):
        col0 = g * 16
        cvec = lane + col0
        rws16 = rws_v[pl.ds(col0, 16)] * jnp.float32(INV_DELTA)
        for j in range(NUM_ATOMS):
            idxf = jnp.clip(jnp.float32(j) + rws16, 0.0, float(TOP_BIN))
            li = idxf.astype(jnp.int32)
            f = idxf - li.astype(jnp.float32)
            ui = jnp.minimum(li + 1, TOP_BIN)
            p = probs_v[j, pl.ds(col0, 16)]
            plsc.addupdate_scatter(out_v, [li, cvec], p * (1.0 - f))
            plsc.addupdate_scatter(out_v, [ui, cvec], p * f)

    pltpu.sync_copy(out_v, out_t_hbm.at[:, pl.ds(base, COLS_PER_W)])


@functools.partial(
    pl.kernel,
    out_type=jax.ShapeDtypeStruct((NUM_ATOMS, BS), jnp.float32),
    mesh=plsc.VectorSubcoreMesh(core_axis_name="c", subcore_axis_name="s"),
    compiler_params=pltpu.CompilerParams(
        needs_layout_passes=False, use_tc_tiling_on_sc=True),
    scratch_types=[
        pltpu.VMEM((COLS_PER_W,), jnp.float32),
        pltpu.VMEM((NUM_ATOMS, COLS_PER_W), jnp.float32),
        pltpu.VMEM((NUM_ATOMS, COLS_PER_W), jnp.float32),
        pltpu.SemaphoreType.DMA,
    ],
)
def _projection_kernel(rw_hbm, probs_t_hbm, out_t_hbm,
                       rws_v, probs_v, out_v, sem):
    _sc_project(rw_hbm, probs_t_hbm, out_t_hbm, rws_v, probs_v, out_v, sem)


def kernel(reward, probs):
    return _projection_kernel(reward, probs.T).T


# speedup vs baseline: 276.7852x; 1.2077x over previous
import functools

import jax
import jax.numpy as jnp
from jax import lax
from jax.experimental import pallas as pl
from jax.experimental.pallas import tpu as pltpu
from jax.experimental.pallas import tpu_sc as plsc

V_MIN = -10.0
V_MAX = 10.0
NUM_ATOMS = 51
ATOM_DELTA = (V_MAX - V_MIN) / (NUM_ATOMS - 1)
INV_DELTA = 1.0 / ATOM_DELTA
BS = 16384
TOP_BIN = NUM_ATOMS - 1

NUM_CORES = 2
NUM_SUBCORES = 16
NUM_WORKERS = NUM_CORES * NUM_SUBCORES
COLS_PER_W = BS // NUM_WORKERS
GROUPS = COLS_PER_W // 16


def _sc_project(rw_hbm, probs_t_hbm, out_t_hbm, rws_v, probs_v, out_v, sem):
    wid = lax.axis_index("s") * NUM_CORES + lax.axis_index("c")
    base = wid * COLS_PER_W
    rw_cp = pltpu.async_copy(rw_hbm.at[pl.ds(base, COLS_PER_W)], rws_v, sem)
    p_cp = pltpu.async_copy(
        probs_t_hbm.at[:, pl.ds(base, COLS_PER_W)], probs_v, sem)

    zeros16 = jnp.zeros((16,), jnp.float32)
    lane = lax.iota(jnp.int32, 16)

    @plsc.parallel_loop(0, NUM_ATOMS, step=1, unroll=2)
    def zero_body(j):
        for g in range(GROUPS):
            out_v[j, pl.ds(g * 16, 16)] = zeros16

    rw_cp.wait()
    p_cp.wait()

    @plsc.parallel_loop(0, GROUPS, step=1, unroll=2)
    def group_body(g):
        col0 = g * 16
        cvec = lane + col0
        s16 = jnp.clip(rws_v[pl.ds(col0, 16)] * jnp.float32(INV_DELTA),
                       -52.0, 52.0)
        t = s16.astype(jnp.int32)
        m16 = t - (t.astype(jnp.float32) > s16).astype(jnp.int32)
        f16 = s16 - m16.astype(jnp.float32)
        omf = 1.0 - f16
        lo = jnp.clip(m16, 0, TOP_BIN)
        for j in range(NUM_ATOMS):
            hi = jnp.clip(m16 + (j + 1), 0, TOP_BIN)
            p = probs_v[j, pl.ds(col0, 16)]
            plsc.addupdate_scatter(out_v, [lo, cvec], p * omf)
            plsc.addupdate_scatter(out_v, [hi, cvec], p * f16)
            lo = hi

    pltpu.sync_copy(out_v, out_t_hbm.at[:, pl.ds(base, COLS_PER_W)])


@functools.partial(
    pl.kernel,
    out_type=jax.ShapeDtypeStruct((NUM_ATOMS, BS), jnp.float32),
    mesh=plsc.VectorSubcoreMesh(core_axis_name="c", subcore_axis_name="s"),
    compiler_params=pltpu.CompilerParams(
        needs_layout_passes=False, use_tc_tiling_on_sc=True),
    scratch_types=[
        pltpu.VMEM((COLS_PER_W,), jnp.float32),
        pltpu.VMEM((NUM_ATOMS, COLS_PER_W), jnp.float32),
        pltpu.VMEM((NUM_ATOMS, COLS_PER_W), jnp.float32),
        pltpu.SemaphoreType.DMA,
    ],
)
def _projection_kernel(rw_hbm, probs_t_hbm, out_t_hbm,
                       rws_v, probs_v, out_v, sem):
    _sc_project(rw_hbm, probs_t_hbm, out_t_hbm, rws_v, probs_v, out_v, sem)


def kernel(reward, probs):
    return _projection_kernel(reward, probs.T).T
